# Initial kernel scaffold; baseline (speedup 1.0000x reference)
import jax
import jax.numpy as jnp
from jax.experimental import pallas as pl


def kernel(x, weights):
    raise NotImplementedError("write your pallas kernel here")


# speedup vs baseline: 5.1975x; 5.1975x over previous
import functools

import jax
import jax.numpy as jnp
import numpy as np
from jax import lax
from jax.experimental import pallas as pl
from jax.experimental.pallas import tpu as pltpu
from jax.experimental.pallas import tpu_sc as plsc

B = 4096
_I0 = np.int32(0)
D = 128
SUB = 8
BW = B // SUB
QB = 512
KB = 512

NC = 2
NS = 16
NW = NC * NS
BPW = B // NW


def _hash_trunc_kernel(xr_ref, x_ref, w1_ref, w2_ref, w3_ref, w4_ref,
                       v_ref, xt_ref):
    xt_ref[...] = x_ref[...].astype(jnp.int32)

    def two_sum(s, c, p):
        t = s + p
        bb = t - s
        e2 = (s - (t - bb)) + (p - bb)
        return t, c + e2

    def body(k, carry):
        s, c = carry
        xk = xr_ref[k]
        w1 = w1_ref[k]
        w2 = w2_ref[k]
        w3 = w3_ref[k]
        w4 = w4_ref[k]
        xb = lax.bitcast_convert_type(xk, jnp.int32)
        xh = lax.bitcast_convert_type(xb & jnp.int32(-4096), jnp.float32)
        xl = xk - xh
        m = (w2 * xl + w3 * xk) + w4 * xk
        s, c = two_sum(s, c, w1 * xh)
        s, c = two_sum(s, c, w1 * xl)
        s, c = two_sum(s, c, w2 * xh)
        s, c = two_sum(s, c, m)
        return s, c

    zero = jnp.zeros((SUB, BW), jnp.float32)
    s, c = lax.fori_loop(0, D, body, (zero, zero))

    c2 = c
    i0 = s.astype(jnp.int32)
    r = s - i0.astype(jnp.float32)
    fh = r + c2
    bb = fh - r
    ferr = (r - (fh - bb)) + (c2 - bb)
    hi = (fh >= 1.0)
    lo = (fh <= -1.0)
    i1 = i0 + hi.astype(jnp.int32) - lo.astype(jnp.int32)
    f3 = (fh - hi.astype(jnp.float32) + lo.astype(jnp.float32)) + ferr
    fl = i1 + jnp.where(f3 < 0.0, jnp.int32(-1), jnp.int32(0))
    neg = (i1 < 0) | ((i1 == 0) & (f3 < 0.0))
    hasf = f3 != 0.0
    v_ref[...] = fl + (neg & hasf).astype(jnp.int32)


def _first_kernel(vrow_ref, vcol_ref, first_ref):
    vq = vcol_ref[...]
    acc = jnp.full((QB, 1), B, jnp.int32)
    for jc in range(B // KB):
        vk = vrow_ref[0:1, jc * KB:(jc + 1) * KB]
        eq = vq == vk
        iota = lax.broadcasted_iota(jnp.int32, (QB, KB), 1) + (jc * KB)
        cand = jnp.where(eq, iota, B)
        acc = jnp.minimum(acc, jnp.min(cand, axis=1, keepdims=True))
    first_ref[...] = acc


def _gather_kernel(xt_hbm, idx_hbm, out_hbm, idx_v, rows_v, sem):
    wid = lax.axis_index("s") * NC + lax.axis_index("c")
    base = wid * BPW
    pltpu.sync_copy(idx_hbm.at[pl.ds(base, BPW)], idx_v)
    pltpu.async_copy(xt_hbm.at[idx_v], rows_v, sem).wait()
    pltpu.sync_copy(rows_v, out_hbm.at[pl.ds(base, BPW)])


def _mask12(a):
    b = lax.bitcast_convert_type(a, jnp.int32)
    return lax.bitcast_convert_type(b & jnp.int32(-4096), jnp.float32)


def kernel(x, weights):
    w1 = _mask12(weights.astype(jnp.float32))
    r1 = weights - w1.astype(jnp.float64)
    w2 = _mask12(r1.astype(jnp.float32))
    r2 = r1 - w2.astype(jnp.float64)
    w3 = _mask12(r2.astype(jnp.float32))
    w4 = (r2 - w3.astype(jnp.float64)).astype(jnp.float32)
    xr = x.T.reshape(D, SUB, BW)

    smem = pl.BlockSpec(memory_space=pltpu.SMEM)
    v8, xt = pl.pallas_call(
        _hash_trunc_kernel,
        in_specs=[pl.BlockSpec((D, SUB, BW), lambda: (_I0, _I0, _I0)),
                  pl.BlockSpec((B, D), lambda: (_I0, _I0)),
                  smem, smem, smem, smem],
        out_specs=[pl.BlockSpec((SUB, BW), lambda: (_I0, _I0)),
                   pl.BlockSpec((B, D), lambda: (_I0, _I0))],
        out_shape=[jax.ShapeDtypeStruct((SUB, BW), jnp.int32),
                   jax.ShapeDtypeStruct((B, D), jnp.int32)],
    )(xr, x, w1, w2, w3, w4)

    vflat = v8.reshape(B)
    first = pl.pallas_call(
        _first_kernel,
        grid=(B // QB,),
        in_specs=[pl.BlockSpec((1, B), lambda i: (_I0, _I0)),
                  pl.BlockSpec((QB, 1), lambda i: (i, _I0))],
        out_specs=pl.BlockSpec((QB, 1), lambda i: (i, _I0)),
        out_shape=jax.ShapeDtypeStruct((B, 1), jnp.int32),
    )(vflat.reshape(1, B), vflat.reshape(B, 1))

    mesh = plsc.VectorSubcoreMesh(core_axis_name="c", subcore_axis_name="s")
    gather = functools.partial(
        pl.kernel,
        out_type=jax.ShapeDtypeStruct((B, D), jnp.int32),
        mesh=mesh,
        scratch_types=[
            pltpu.VMEM((BPW,), jnp.int32),
            pltpu.VMEM((BPW, D), jnp.int32),
            pltpu.SemaphoreType.DMA,
        ],
    )(_gather_kernel)
    return gather(xt, first.reshape(B))
